# baseline (speedup 1.0000x reference)
import jax
import jax.numpy as jnp
from jax import lax
from jax.experimental import pallas as pl
from jax.experimental.pallas import tpu as pltpu
from jax.experimental.pallas import tpu_sc as plsc

B = 16384
_info = plsc.get_sparse_core_info()
NC, NS, L = _info.num_cores, _info.num_subcores, _info.num_lanes
NW = NC * NS
BPW = B // NW

_MESH = plsc.VectorSubcoreMesh(core_axis_name="c", subcore_axis_name="s")


def _gather_u(users_hbm, ub_hbm, pu_hbm, idx, v, sem):
    wid = lax.axis_index("s") * NC + lax.axis_index("c")
    base = wid * BPW
    pltpu.sync_copy(users_hbm.at[pl.ds(base, BPW)], idx)
    pltpu.async_copy(ub_hbm.at[idx], v, sem).wait()
    pltpu.sync_copy(v, pu_hbm.at[pl.ds(base, BPW)])


def _gather_i_add(items_hbm, ib_hbm, pu_hbm, out_hbm, idx, v, u, sem, sem_u):
    wid = lax.axis_index("s") * NC + lax.axis_index("c")
    base = wid * BPW
    pltpu.sync_copy(items_hbm.at[pl.ds(base, BPW)], idx)
    cu = pltpu.async_copy(pu_hbm.at[pl.ds(base, BPW)], u, sem_u)
    pltpu.async_copy(ib_hbm.at[idx], v, sem).wait()
    cu.wait()
    for j in range(BPW // L):
        sl = pl.ds(j * L, L)
        u[sl] = u[sl] + v[sl]
    pltpu.sync_copy(u, out_hbm.at[pl.ds(base, BPW)])


def kernel(users, items, user_bias, item_bias):
    ub = user_bias.reshape(-1)
    ib = item_bias.reshape(-1)
    run_u = pl.kernel(
        _gather_u,
        out_type=jax.ShapeDtypeStruct((B,), jnp.float32),
        mesh=_MESH,
        scratch_types=[
            pltpu.VMEM((BPW,), jnp.int32),
            pltpu.VMEM((BPW,), jnp.float32),
            pltpu.SemaphoreType.DMA,
        ],
    )
    pu = run_u(users, ub)
    run_i = pl.kernel(
        _gather_i_add,
        out_type=jax.ShapeDtypeStruct((B,), jnp.float32),
        mesh=_MESH,
        scratch_types=[
            pltpu.VMEM((BPW,), jnp.int32),
            pltpu.VMEM((BPW,), jnp.float32),
            pltpu.VMEM((BPW,), jnp.float32),
            pltpu.SemaphoreType.DMA,
            pltpu.SemaphoreType.DMA,
        ],
    )
    return run_i(items, ib, pu)


# speedup vs baseline: 3.0749x; 3.0749x over previous
import jax
import jax.numpy as jnp
from jax import lax
from jax.experimental import pallas as pl
from jax.experimental.pallas import tpu as pltpu
from jax.experimental.pallas import tpu_sc as plsc

B = 16384
_info = plsc.get_sparse_core_info()
NC, NS, L = _info.num_cores, _info.num_subcores, _info.num_lanes
NW = NC * NS
BPW = B // NW

_MESH = plsc.VectorSubcoreMesh(core_axis_name="c", subcore_axis_name="s")


def _body(users_hbm, items_hbm, ub_hbm, ib_hbm, out_hbm,
          idx_u, idx_i, u1, i1, sem_u, sem_i):
    wid = lax.axis_index("s") * NC + lax.axis_index("c")
    base = wid * BPW
    pltpu.sync_copy(users_hbm.at[pl.ds(base, BPW)], idx_u)
    pltpu.sync_copy(items_hbm.at[pl.ds(base, BPW)], idx_i)
    cu = pltpu.async_copy(ub_hbm.at[idx_u], u1, sem_u)
    ci = pltpu.async_copy(ib_hbm.at[idx_i], i1, sem_i)
    cu.wait()
    ci.wait()
    for j in range(BPW // L):
        sl = pl.ds(j * L, L)
        u1[sl] = u1[sl] + i1[sl]
    pltpu.sync_copy(u1, out_hbm.at[pl.ds(base, BPW)])


def kernel(users, items, user_bias, item_bias):
    ub1 = jnp.pad(user_bias, ((0, 448), (0, 0))).reshape(-1)
    ib1 = jnp.pad(item_bias, ((0, 448), (0, 0))).reshape(-1)
    run = pl.kernel(
        _body,
        out_type=jax.ShapeDtypeStruct((B,), jnp.float32),
        mesh=_MESH,
        scratch_types=[
            pltpu.VMEM((BPW,), jnp.int32),
            pltpu.VMEM((BPW,), jnp.int32),
            pltpu.VMEM((BPW,), jnp.float32),
            pltpu.VMEM((BPW,), jnp.float32),
            pltpu.SemaphoreType.DMA,
            pltpu.SemaphoreType.DMA,
        ],
    )
    return run(users, items, ub1, ib1)


# speedup vs baseline: 3.0811x; 1.0020x over previous
import jax
import jax.numpy as jnp
from jax import lax
from jax.experimental import pallas as pl
from jax.experimental.pallas import tpu as pltpu
from jax.experimental.pallas import tpu_sc as plsc

B = 16384
_info = plsc.get_sparse_core_info()
NC, NS, L = _info.num_cores, _info.num_subcores, _info.num_lanes
NW = NC * NS
BPW = B // NW

_MESH = plsc.VectorSubcoreMesh(core_axis_name="c", subcore_axis_name="s")


def _body(users_hbm, items_hbm, ub_hbm, ib_hbm, out_hbm,
          idx_u, idx_i, u1, i1, sem_u, sem_i):
    wid = lax.axis_index("s") * NC + lax.axis_index("c")
    base = wid * BPW
    pltpu.sync_copy(users_hbm.at[pl.ds(base, BPW)], idx_u)
    pltpu.sync_copy(items_hbm.at[pl.ds(base, BPW)], idx_i)
    cu = pltpu.async_copy(ub_hbm.at[idx_u], u1, sem_u)
    ci = pltpu.async_copy(ib_hbm.at[idx_i], i1, sem_i)
    cu.wait()
    ci.wait()
    for j in range(BPW // L):
        sl = pl.ds(j * L, L)
        u1[sl] = u1[sl] + i1[sl]
    pltpu.sync_copy(u1, out_hbm.at[pl.ds(base, BPW)])


def kernel(users, items, user_bias, item_bias):
    ub1 = jnp.pad(user_bias, ((0, 448), (0, 0))).reshape(-1)
    ib1 = jnp.pad(item_bias, ((0, 448), (0, 0))).reshape(-1)
    run = pl.kernel(
        _body,
        out_type=jax.ShapeDtypeStruct((B,), jnp.float32),
        mesh=_MESH,
        compiler_params=pltpu.CompilerParams(skip_device_barrier=True),
        scratch_types=[
            pltpu.VMEM((BPW,), jnp.int32),
            pltpu.VMEM((BPW,), jnp.int32),
            pltpu.VMEM((BPW,), jnp.float32),
            pltpu.VMEM((BPW,), jnp.float32),
            pltpu.SemaphoreType.DMA,
            pltpu.SemaphoreType.DMA,
        ],
    )
    return run(users, items, ub1, ib1)


# speedup vs baseline: 3.1257x; 1.0145x over previous
import jax
import jax.numpy as jnp
from jax import lax
from jax.experimental import pallas as pl
from jax.experimental.pallas import tpu as pltpu
from jax.experimental.pallas import tpu_sc as plsc

B = 16384
_info = plsc.get_sparse_core_info()
NC, NS, L = _info.num_cores, _info.num_subcores, _info.num_lanes
NW = NC * NS
BPW = B // NW

_MESH = plsc.VectorSubcoreMesh(core_axis_name="c", subcore_axis_name="s")


def _body(users_hbm, items_hbm, ub_hbm, ib_hbm, out_hbm,
          idx_u, idx_i, u1, i1, sem_u, sem_i):
    wid = lax.axis_index("s") * NC + lax.axis_index("c")
    base = wid * BPW
    cxu = pltpu.async_copy(users_hbm.at[pl.ds(base, BPW)], idx_u, sem_u)
    cxi = pltpu.async_copy(items_hbm.at[pl.ds(base, BPW)], idx_i, sem_i)
    cxu.wait()
    cu = pltpu.async_copy(ub_hbm.at[idx_u], u1, sem_u)
    cxi.wait()
    ci = pltpu.async_copy(ib_hbm.at[idx_i], i1, sem_i)
    cu.wait()
    ci.wait()
    for j in range(BPW // L):
        sl = pl.ds(j * L, L)
        u1[sl] = u1[sl] + i1[sl]
    pltpu.sync_copy(u1, out_hbm.at[pl.ds(base, BPW)])


def kernel(users, items, user_bias, item_bias):
    ub1 = jnp.pad(user_bias, ((0, 448), (0, 0))).reshape(-1)
    ib1 = jnp.pad(item_bias, ((0, 448), (0, 0))).reshape(-1)
    run = pl.kernel(
        _body,
        out_type=jax.ShapeDtypeStruct((B,), jnp.float32),
        mesh=_MESH,
        scratch_types=[
            pltpu.VMEM((BPW,), jnp.int32),
            pltpu.VMEM((BPW,), jnp.int32),
            pltpu.VMEM((BPW,), jnp.float32),
            pltpu.VMEM((BPW,), jnp.float32),
            pltpu.SemaphoreType.DMA,
            pltpu.SemaphoreType.DMA,
        ],
    )
    return run(users, items, ub1, ib1)
